# Initial kernel scaffold; baseline (speedup 1.0000x reference)
import functools

import jax
import jax.numpy as jnp
from jax import lax
from jax.experimental import pallas as pl
from jax.experimental.pallas import tpu as pltpu
from jax.experimental.pallas import tpu_sc as plsc

D = 64
K = 20
L = 16


@functools.lru_cache(maxsize=None)
def _build(B):
    info = plsc.get_sparse_core_info()
    NC, NS = info.num_cores, info.num_subcores
    NW = NC * NS
    BPW = B // NW
    C = 32
    NCH = BPW // C
    GS = 128
    NSL = (C * K) // GS
    mesh = plsc.VectorSubcoreMesh(core_axis_name="c", subcore_axis_name="s")

    @functools.partial(
        pl.kernel, mesh=mesh,
        out_type=[jax.ShapeDtypeStruct((B,), jnp.float32),
                  jax.ShapeDtypeStruct((B * K,), jnp.float32)],
        scratch_types=[
            pltpu.VMEM((C,), jnp.int32),
            pltpu.VMEM((C,), jnp.int32),
            pltpu.VMEM((C * K,), jnp.int32),
            pltpu.VMEM((C, D), jnp.float32),
            pltpu.VMEM((C, D), jnp.float32),
            pltpu.VMEM((C * K, D), jnp.float32),
            pltpu.VMEM((C,), jnp.float32),
            pltpu.VMEM((C * K,), jnp.float32),
            pltpu.SemaphoreType.DMA,
        ],
    )
    def sc_kernel(in_lab, pos_lab, neg_lab, in_tab, out_tab,
                  pos_out, neg_out,
                  in_idx, pos_idx, neg_idx, in_rows, pos_rows, neg_rows,
                  pos_buf, neg_buf, sem):
        wid = lax.axis_index("s") * NC + lax.axis_index("c")
        base = wid * BPW
        lanes = lax.iota(jnp.int32, L)

        def chunk(c, carry):
            off = base + c * C
            pltpu.sync_copy(in_lab.at[pl.ds(off, C)], in_idx)
            pltpu.sync_copy(pos_lab.at[pl.ds(off, C)], pos_idx)
            pltpu.sync_copy(neg_lab.at[pl.ds(off * K, C * K)], neg_idx)
            cps = [pltpu.async_copy(in_tab.at[in_idx], in_rows, sem),
                   pltpu.async_copy(out_tab.at[pos_idx], pos_rows, sem)]
            for j in range(NSL):
                cps.append(pltpu.async_copy(
                    out_tab.at[neg_idx.at[pl.ds(j * GS, GS)]],
                    neg_rows.at[pl.ds(j * GS, GS)], sem))
            for cp in cps:
                cp.wait()

            for g in range(C // L):
                gl = g * L + lanes

                def dstep(d, accs):
                    dcol = jnp.full((L,), d, jnp.int32)
                    iv = plsc.load_gather(in_rows, [gl, dcol])
                    pv = plsc.load_gather(pos_rows, [gl, dcol])
                    out = [accs[0] + iv * pv]
                    for kk in range(K):
                        nv = plsc.load_gather(neg_rows, [gl * K + kk, dcol])
                        out.append(accs[kk + 1] + nv * iv)
                    return tuple(out)

                zero = jnp.zeros((L,), jnp.float32)
                accs = lax.fori_loop(0, D, dstep, (zero,) * (K + 1))
                pos_buf[pl.ds(g * L, L)] = accs[0]
                for kk in range(K):
                    plsc.store_scatter(neg_buf, [gl * K + kk], accs[kk + 1])

            pltpu.sync_copy(pos_buf, pos_out.at[pl.ds(off, C)])
            pltpu.sync_copy(neg_buf, neg_out.at[pl.ds(off * K, C * K)])
            return carry

        lax.fori_loop(0, NCH, chunk, 0)

    return sc_kernel


def kernel(input_labels, pos_labels, neg_labels, in_table, out_table):
    B = input_labels.shape[0]
    il = input_labels.astype(jnp.int32)
    pli = pos_labels.astype(jnp.int32)
    nli = neg_labels.astype(jnp.int32).reshape(-1)
    sc = _build(B)
    pos_flat, neg_flat = sc(il, pli, nli, in_table, out_table)
    return pos_flat.reshape(B, 1), neg_flat.reshape(B, neg_labels.shape[1])


# speedup vs baseline: 4.0460x; 4.0460x over previous
import functools

import jax
import jax.numpy as jnp
from jax import lax
from jax.experimental import pallas as pl
from jax.experimental.pallas import tpu as pltpu
from jax.experimental.pallas import tpu_sc as plsc

D = 64
K = 20
L = 16
C = 32
GS = 128


@functools.lru_cache(maxsize=None)
def _build(B):
    info = plsc.get_sparse_core_info()
    NC, NS = info.num_cores, info.num_subcores
    NW = NC * NS
    BPW = B // NW
    NCH = BPW // C
    NSL = (C * K) // GS
    mesh = plsc.VectorSubcoreMesh(core_axis_name="c", subcore_axis_name="s")

    buf_set = [
        pltpu.VMEM((C,), jnp.int32),
        pltpu.VMEM((C,), jnp.int32),
        pltpu.VMEM((C * K,), jnp.int32),
        pltpu.VMEM((C, D), jnp.float32),
        pltpu.VMEM((C, D), jnp.float32),
        pltpu.VMEM((C * K, D), jnp.float32),
        pltpu.VMEM((C,), jnp.float32),
        pltpu.VMEM((C * K,), jnp.float32),
        pltpu.SemaphoreType.DMA,
        pltpu.SemaphoreType.DMA,
        pltpu.SemaphoreType.DMA,
    ]

    @functools.partial(
        pl.kernel, mesh=mesh,
        compiler_params=pltpu.CompilerParams(
            needs_layout_passes=False, use_tc_tiling_on_sc=False),
        out_type=[jax.ShapeDtypeStruct((B,), jnp.float32),
                  jax.ShapeDtypeStruct((B * K,), jnp.float32)],
        scratch_types=buf_set + buf_set,
    )
    def sc_kernel(in_lab, pos_lab, neg_lab, in_tab, out_tab,
                  pos_out, neg_out, *scratch):
        wid = lax.axis_index("s") * NC + lax.axis_index("c")
        base = wid * BPW
        lanes = lax.iota(jnp.int32, L)
        ph = (scratch[:11], scratch[11:])

        def bufs(p):
            (in_idx, pos_idx, neg_idx, in_rows, pos_rows, neg_rows,
             pos_buf, neg_buf, sem_i, sem_g, sem_o) = ph[p]
            return dict(in_idx=in_idx, pos_idx=pos_idx, neg_idx=neg_idx,
                        in_rows=in_rows, pos_rows=pos_rows,
                        neg_rows=neg_rows, pos_buf=pos_buf, neg_buf=neg_buf,
                        sem_i=sem_i, sem_g=sem_g, sem_o=sem_o)

        def idx_copies(c, p):
            b = bufs(p)
            off = base + c * C
            return [
                pltpu.make_async_copy(in_lab.at[pl.ds(off, C)], b["in_idx"],
                                      b["sem_i"]),
                pltpu.make_async_copy(pos_lab.at[pl.ds(off, C)], b["pos_idx"],
                                      b["sem_i"]),
                pltpu.make_async_copy(neg_lab.at[pl.ds(off * K, C * K)],
                                      b["neg_idx"], b["sem_i"]),
            ]

        def gather_copies(p):
            b = bufs(p)
            cps = [
                pltpu.make_async_copy(in_tab.at[b["in_idx"]], b["in_rows"],
                                      b["sem_g"]),
                pltpu.make_async_copy(out_tab.at[b["pos_idx"]], b["pos_rows"],
                                      b["sem_g"]),
            ]
            for j in range(NSL):
                cps.append(pltpu.make_async_copy(
                    out_tab.at[b["neg_idx"].at[pl.ds(j * GS, GS)]],
                    b["neg_rows"].at[pl.ds(j * GS, GS)], b["sem_g"]))
            return cps

        def out_copies(c, p):
            b = bufs(p)
            off = base + c * C
            return [
                pltpu.make_async_copy(b["pos_buf"],
                                      pos_out.at[pl.ds(off, C)], b["sem_o"]),
                pltpu.make_async_copy(b["neg_buf"],
                                      neg_out.at[pl.ds(off * K, C * K)],
                                      b["sem_o"]),
            ]

        def fire(cps):
            for cp in cps:
                cp.start()

        def drain(cps):
            for cp in cps:
                cp.wait()

        def compute(p):
            b = bufs(p)
            in_rows, pos_rows, neg_rows = (
                b["in_rows"], b["pos_rows"], b["neg_rows"])
            for g in range(C // L):
                gl = g * L + lanes
                glk = gl * K

                def dstep(d, accs):
                    dcol = jnp.full((L,), d, jnp.int32)
                    iv = plsc.load_gather(in_rows, [gl, dcol])
                    pv = plsc.load_gather(pos_rows, [gl, dcol])
                    out = [accs[0] + iv * pv]
                    for kk in range(K):
                        nv = plsc.load_gather(neg_rows, [glk + kk, dcol])
                        out.append(accs[kk + 1] + nv * iv)
                    return tuple(out)

                zero = jnp.zeros((L,), jnp.float32)
                accs = lax.fori_loop(0, D, dstep, (zero,) * (K + 1))
                b["pos_buf"][pl.ds(g * L, L)] = accs[0]
                for kk in range(K):
                    plsc.store_scatter(b["neg_buf"], [glk + kk], accs[kk + 1])

        fire(idx_copies(0, 0))
        fire(idx_copies(1, 1))
        drain(idx_copies(0, 0))
        fire(gather_copies(0))

        def body(s, carry):
            A = 2 * s
            Bc = A + 1
            drain(idx_copies(Bc, 1))
            fire(gather_copies(1))
            drain(gather_copies(0))

            @pl.when(s > 0)
            def _():
                drain(out_copies(A - 2, 0))

            compute(0)
            fire(out_copies(A, 0))

            @pl.when(s < NCH // 2 - 1)
            def _():
                fire(idx_copies(A + 2, 0))
                drain(idx_copies(A + 2, 0))
                fire(gather_copies(0))

            drain(gather_copies(1))

            @pl.when(s > 0)
            def _():
                drain(out_copies(Bc - 2, 1))

            compute(1)
            fire(out_copies(Bc, 1))

            @pl.when(s < NCH // 2 - 1)
            def _():
                fire(idx_copies(Bc + 2, 1))

            return carry

        lax.fori_loop(0, NCH // 2, body, 0)
        drain(out_copies(NCH - 2, 0))
        drain(out_copies(NCH - 1, 1))

    return sc_kernel


def kernel(input_labels, pos_labels, neg_labels, in_table, out_table):
    B = input_labels.shape[0]
    il = input_labels.astype(jnp.int32)
    pli = pos_labels.astype(jnp.int32)
    nli = neg_labels.astype(jnp.int32).reshape(-1)
    sc = _build(B)
    pos_flat, neg_flat = sc(il, pli, nli, in_table, out_table)
    return pos_flat.reshape(B, 1), neg_flat.reshape(B, neg_labels.shape[1])
